# Initial kernel scaffold; baseline (speedup 1.0000x reference)
import jax
import jax.numpy as jnp
from jax.experimental import pallas as pl


def kernel(xyz, t, batch, spatial_emb, temporal_emb, bounds):
    raise NotImplementedError("write your pallas kernel here")


# speedup vs baseline: 103.7663x; 103.7663x over previous
import functools

import jax
import jax.numpy as jnp
from jax import lax
from jax.experimental import pallas as pl
from jax.experimental.pallas import tpu as pltpu
from jax.experimental.pallas import tpu_sc as plsc

NC = 2
NS = 16
NW = NC * NS
C = 128


def _patch_rows(img):
    h, w, f = img.shape
    r1 = jnp.roll(img, -1, axis=1)
    r2 = jnp.roll(img, -1, axis=0)
    r3 = jnp.roll(r2, -1, axis=1)
    return jnp.concatenate([img, r1, r2, r3], axis=-1).reshape(h * w, 4 * f)


def _build_pairs(spatial_emb, temporal_emb):
    res = [spatial_emb[i].shape[-1] for i in range(len(spatial_emb))]
    parts = []
    pairs = []
    off = 0
    sel = [(0, 1), (0, 2), (1, 2)]
    for g in range(3):
        a, b = sel[g]
        for l in range(len(res)):
            r = res[l]
            img = jnp.transpose(spatial_emb[l][g], (1, 2, 0))
            parts.append(_patch_rows(img))
            pairs.append(dict(a=a, b=b, sx=float(r), cx=-0.5,
                              sy=float(r), cy=-0.5, w=r, h=r,
                              off=off, col=g * 8 + l * 2))
            off += r * r
    for g in range(3):
        for l in range(len(res)):
            r = res[l]
            img = jnp.transpose(temporal_emb[l][g], (1, 2, 0))
            dim_t = img.shape[1]
            parts.append(_patch_rows(img))
            pairs.append(dict(a=3, b=g, sx=float(dim_t - 1), cx=0.0,
                              sy=float(r - 1), cy=0.0, w=dim_t, h=r,
                              off=off, col=(3 + g) * 8 + l * 2))
            off += r * dim_t
    table = jnp.concatenate(parts, axis=0)
    return table.reshape(-1), pairs


def _sc_kernel(pairs, n_points, coords, table):
    npairs = len(pairs)
    ppw = n_points // NW
    nch = ppw // C
    nf = 2 * npairs

    mesh = plsc.VectorSubcoreMesh(core_axis_name="c", subcore_axis_name="s")

    @functools.partial(
        pl.kernel,
        out_type=jax.ShapeDtypeStruct((n_points // C, nf, C), jnp.float32),
        mesh=mesh,
        scratch_types=[
            pltpu.VMEM((4, C), jnp.float32),
            pltpu.VMEM((npairs * 8, C), jnp.int32),
            pltpu.VMEM((npairs, C), jnp.float32),
            pltpu.VMEM((npairs, C), jnp.float32),
            [pltpu.VMEM((8, C), jnp.float32) for _ in range(npairs)],
            pltpu.VMEM((nf, C), jnp.float32),
            pltpu.SemaphoreType.DMA,
            pltpu.SemaphoreType.DMA,
        ],
    )
    def run(coords_hbm, table_hbm, out_hbm,
            coords_v, idx_v, wx_v, wy_v, gath_v, out_v, csem, gsem):
        wid = lax.axis_index("s") * NC + lax.axis_index("c")
        base0 = wid * ppw

        def chunk_body(ch, carry):
            base = base0 + ch * C
            cid = wid * nch + ch
            cds = [pltpu.async_copy(coords_hbm.at[k, pl.ds(base, C)],
                                    coords_v.at[k], csem)
                   for k in range(4)]
            for d in cds:
                d.wait()

            descs = []
            for p, prm in enumerate(pairs):
                a_row, b_row = prm["a"], prm["b"]
                sx, cx = prm["sx"], prm["cx"]
                sy, cy = prm["sy"], prm["cy"]
                w, h, off = prm["w"], prm["h"], prm["off"]

                def istep(i, c2, a_row=a_row, b_row=b_row, sx=sx, cx=cx,
                          sy=sy, cy=cy, w=w, h=h, off=off, p=p):
                    sl = pl.ds(i * 16, 16)
                    av = coords_v[a_row, sl]
                    bv = coords_v[b_row, sl]
                    ix = av * sx + cx
                    iy = bv * sy + cy
                    ui = jnp.clip(ix.astype(jnp.int32), 0, w - 2)
                    vi = jnp.clip(iy.astype(jnp.int32), 0, h - 2)
                    wx = jnp.clip(ix - ui.astype(jnp.float32), 0.0, 1.0)
                    wy = jnp.clip(iy - vi.astype(jnp.float32), 0.0, 1.0)
                    fidx = (vi * w + ui + off) * 8
                    for k in range(8):
                        idx_v[p * 8 + k, sl] = fidx + k
                    wx_v[p, sl] = wx
                    wy_v[p, sl] = wy
                    return c2

                lax.fori_loop(0, C // 16, istep, 0)
                for k in range(8):
                    descs.append(pltpu.async_copy(
                        table_hbm.at[idx_v.at[p * 8 + k]],
                        gath_v[p].at[k], gsem))

            for p, prm in enumerate(pairs):
                for d in descs[p * 8:(p + 1) * 8]:
                    d.wait()
                col = prm["col"]

                def bstep(i, c2, p=p, col=col):
                    sl = pl.ds(i * 16, 16)
                    wx1 = wx_v[p, sl]
                    wy1 = wy_v[p, sl]
                    wx0 = 1.0 - wx1
                    wy0 = 1.0 - wy1
                    w00 = wx0 * wy0
                    w01 = wx1 * wy0
                    w10 = wx0 * wy1
                    w11 = wx1 * wy1
                    g = gath_v[p]
                    cs = [g[k, sl] for k in range(8)]
                    f0 = w00 * cs[0] + w01 * cs[2] + w10 * cs[4] + w11 * cs[6]
                    f1 = w00 * cs[1] + w01 * cs[3] + w10 * cs[5] + w11 * cs[7]
                    out_v[col, sl] = f0
                    out_v[col + 1, sl] = f1
                    return c2

                lax.fori_loop(0, C // 16, bstep, 0)

            pltpu.sync_copy(out_v, out_hbm.at[cid])
            return carry

        lax.fori_loop(0, nch, chunk_body, 0)

    return run(coords, table)


def kernel(xyz, t, batch, spatial_emb, temporal_emb, bounds):
    bash = xyz.shape
    xyz = xyz.reshape(-1, xyz.shape[-1])
    t = t.reshape(-1, t.shape[-1])
    n = xyz.shape[0]
    xyzn = (xyz - bounds[0]) / (bounds[1] - bounds[0])
    coords = jnp.concatenate([xyzn.T, t[:, :1].T], axis=0)
    table, pairs = _build_pairs(spatial_emb, temporal_emb)
    out = _sc_kernel(pairs, n, coords, table)
    out = out.transpose(0, 2, 1).reshape(n, -1)
    return out.reshape(*bash[:-1], out.shape[-1])


# speedup vs baseline: 193.4970x; 1.8647x over previous
import functools

import jax
import jax.numpy as jnp
from jax import lax
from jax.experimental import pallas as pl
from jax.experimental.pallas import tpu as pltpu
from jax.experimental.pallas import tpu_sc as plsc

NC = 2
NS = 16
NW = NC * NS
C = 128


def _patch_rows(img):
    h, w, f = img.shape
    r1 = jnp.roll(img, -1, axis=1)
    r2 = jnp.roll(img, -1, axis=0)
    r3 = jnp.roll(r2, -1, axis=1)
    return jnp.concatenate([img, r1, r2, r3], axis=-1).reshape(h * w, 4 * f)


def _build_pairs(spatial_emb, temporal_emb):
    res = [spatial_emb[i].shape[-1] for i in range(len(spatial_emb))]
    parts = []
    pairs = []
    off = 0
    sel = [(0, 1), (0, 2), (1, 2)]
    for g in range(3):
        a, b = sel[g]
        for l in range(len(res)):
            r = res[l]
            img = jnp.transpose(spatial_emb[l][g], (1, 2, 0))
            parts.append(_patch_rows(img))
            pairs.append(dict(a=a, b=b, sx=float(r), cx=-0.5,
                              sy=float(r), cy=-0.5, w=r, h=r,
                              off=off, col=g * 8 + l * 2))
            off += r * r
    for g in range(3):
        for l in range(len(res)):
            r = res[l]
            img = jnp.transpose(temporal_emb[l][g], (1, 2, 0))
            dim_t = img.shape[1]
            parts.append(_patch_rows(img))
            pairs.append(dict(a=3, b=g, sx=float(dim_t - 1), cx=0.0,
                              sy=float(r - 1), cy=0.0, w=dim_t, h=r,
                              off=off, col=(3 + g) * 8 + l * 2))
            off += r * dim_t
    table = jnp.concatenate(parts, axis=0)
    return table, pairs


def _sc_kernel(pairs, n_points, coords, table):
    npairs = len(pairs)
    ppw = n_points // NW
    nch = ppw // C
    nf = 2 * npairs

    mesh = plsc.VectorSubcoreMesh(core_axis_name="c", subcore_axis_name="s")

    @functools.partial(
        pl.kernel,
        out_type=jax.ShapeDtypeStruct((n_points // C, nf, C), jnp.float32),
        mesh=mesh,
        compiler_params=pltpu.CompilerParams(needs_layout_passes=False, use_tc_tiling_on_sc=False),
        scratch_types=[
            pltpu.VMEM((4, C), jnp.float32),
            pltpu.VMEM((npairs, C), jnp.int32),
            pltpu.VMEM((npairs, C), jnp.float32),
            pltpu.VMEM((npairs, C), jnp.float32),
            [pltpu.VMEM((C, 8), jnp.float32) for _ in range(npairs)],
            pltpu.VMEM((nf, C), jnp.float32),
            pltpu.SemaphoreType.DMA,
            pltpu.SemaphoreType.DMA,
        ],
    )
    def run(coords_hbm, table_hbm, out_hbm,
            coords_v, idx_v, wx_v, wy_v, gath_v, out_v, csem, gsem):
        wid = lax.axis_index("s") * NC + lax.axis_index("c")
        base0 = wid * ppw

        def chunk_body(ch, carry):
            base = base0 + ch * C
            cid = wid * nch + ch
            cds = [pltpu.async_copy(coords_hbm.at[k, pl.ds(base, C)],
                                    coords_v.at[k], csem)
                   for k in range(4)]
            for d in cds:
                d.wait()

            descs = []
            for p, prm in enumerate(pairs):
                a_row, b_row = prm["a"], prm["b"]
                sx, cx = prm["sx"], prm["cx"]
                sy, cy = prm["sy"], prm["cy"]
                w, h, off = prm["w"], prm["h"], prm["off"]

                def istep(i, c2, a_row=a_row, b_row=b_row, sx=sx, cx=cx,
                          sy=sy, cy=cy, w=w, h=h, off=off, p=p):
                    sl = pl.ds(i * 16, 16)
                    av = coords_v[a_row, sl]
                    bv = coords_v[b_row, sl]
                    ix = av * sx + cx
                    iy = bv * sy + cy
                    ui = jnp.clip(ix.astype(jnp.int32), 0, w - 2)
                    vi = jnp.clip(iy.astype(jnp.int32), 0, h - 2)
                    wx = jnp.clip(ix - ui.astype(jnp.float32), 0.0, 1.0)
                    wy = jnp.clip(iy - vi.astype(jnp.float32), 0.0, 1.0)
                    idx_v[p, sl] = vi * w + ui + off
                    wx_v[p, sl] = wx
                    wy_v[p, sl] = wy
                    return c2

                lax.fori_loop(0, C // 16, istep, 0)
                descs.append(pltpu.async_copy(
                    table_hbm.at[idx_v.at[p]], gath_v[p], gsem))

            for p, prm in enumerate(pairs):
                descs[p].wait()
                col = prm["col"]

                def bstep(i, c2, p=p, col=col):
                    sl = pl.ds(i * 16, 16)
                    rows = lax.iota(jnp.int32, 16) + i * 16
                    wx1 = wx_v[p, sl]
                    wy1 = wy_v[p, sl]
                    wx0 = 1.0 - wx1
                    wy0 = 1.0 - wy1
                    w00 = wx0 * wy0
                    w01 = wx1 * wy0
                    w10 = wx0 * wy1
                    w11 = wx1 * wy1
                    g = gath_v[p]
                    cs = [plsc.load_gather(
                        g, [rows, jnp.full((16,), k, jnp.int32)])
                        for k in range(8)]
                    f0 = w00 * cs[0] + w01 * cs[2] + w10 * cs[4] + w11 * cs[6]
                    f1 = w00 * cs[1] + w01 * cs[3] + w10 * cs[5] + w11 * cs[7]
                    out_v[col, sl] = f0
                    out_v[col + 1, sl] = f1
                    return c2

                lax.fori_loop(0, C // 16, bstep, 0)

            pltpu.sync_copy(out_v, out_hbm.at[cid])
            return carry

        lax.fori_loop(0, nch, chunk_body, 0)

    return run(coords, table)


def kernel(xyz, t, batch, spatial_emb, temporal_emb, bounds):
    bash = xyz.shape
    xyz = xyz.reshape(-1, xyz.shape[-1])
    t = t.reshape(-1, t.shape[-1])
    n = xyz.shape[0]
    xyzn = (xyz - bounds[0]) / (bounds[1] - bounds[0])
    coords = jnp.concatenate([xyzn.T, t[:, :1].T], axis=0)
    table, pairs = _build_pairs(spatial_emb, temporal_emb)
    out = _sc_kernel(pairs, n, coords, table)
    out = out.transpose(0, 2, 1).reshape(n, -1)
    return out.reshape(*bash[:-1], out.shape[-1])
